# Initial kernel scaffold; baseline (speedup 1.0000x reference)
import functools

import jax
import jax.numpy as jnp
from jax import lax
from jax.experimental import pallas as pl
from jax.experimental.pallas import tpu as pltpu
from jax.experimental.pallas import tpu_sc as plsc

NUM_TOKEN = 100000
EMBED_DIM = 64
GRID_SIZE = 200
B, N = 4096, 200
R = B * N

NW = 32
ROWS_PER_W = R // NW
CHUNK = 512
NG = CHUNK // 128
NCHUNK = ROWS_PER_W // CHUNK
LANES = 16


@functools.partial(
    pl.kernel,
    out_type=jax.ShapeDtypeStruct((R, EMBED_DIM), jnp.float32),
    mesh=plsc.VectorSubcoreMesh(core_axis_name="c", subcore_axis_name="s"),
    scratch_types=[
        pltpu.VMEM((CHUNK,), jnp.int32),
        pltpu.VMEM((NG, 128), jnp.int32),
        pltpu.VMEM((NG, 128), jnp.int32),
        pltpu.VMEM((CHUNK, EMBED_DIM), jnp.float32),
        pltpu.VMEM((CHUNK, EMBED_DIM), jnp.float32),
        pltpu.SemaphoreType.DMA,
    ],
)
def _embed_kernel(x_hbm, tok_hbm, pos_hbm, out_hbm,
                  x_v, tidx_v, pidx_v, trows_v, prows_v, sem):
    wid = lax.axis_index("s") * 2 + lax.axis_index("c")
    base = wid * ROWS_PER_W

    def chunk_body(ci, carry):
        row0 = base + ci * CHUNK
        pltpu.sync_copy(x_hbm.at[pl.ds(row0, CHUNK)], x_v)
        for g in range(CHUNK // LANES):
            a, off = g // 8, (g % 8) * LANES
            xv = x_v[pl.ds(g * LANES, LANES)]
            rowv = row0 + g * LANES + lax.iota(jnp.int32, LANES)
            nv = lax.rem(rowv, N)
            pad = xv == 0
            tidx_v[a, pl.ds(off, LANES)] = jnp.where(
                pad, nv + (NUM_TOKEN - GRID_SIZE), xv)
            pidx_v[a, pl.ds(off, LANES)] = jnp.where(pad, GRID_SIZE, nv)
        copies = []
        for a in range(NG):
            copies.append(pltpu.async_copy(
                tok_hbm.at[tidx_v.at[a]],
                trows_v.at[pl.ds(a * 128, 128)], sem))
            copies.append(pltpu.async_copy(
                pos_hbm.at[pidx_v.at[a]],
                prows_v.at[pl.ds(a * 128, 128)], sem))
        for cp in copies:
            cp.wait()

        def add_body(j, c2):
            for cc in range(EMBED_DIM // LANES):
                sl = pl.ds(cc * LANES, LANES)
                trows_v[j, sl] = trows_v[j, sl] + prows_v[j, sl]
            return c2

        lax.fori_loop(0, CHUNK, add_body, 0)
        pltpu.sync_copy(trows_v, out_hbm.at[pl.ds(row0, CHUNK)])
        return carry

    lax.fori_loop(0, NCHUNK, chunk_body, 0)


def kernel(x, tok_embed, pos_embed):
    out = _embed_kernel(x.reshape(R), tok_embed, pos_embed)
    return out.reshape(B, N, EMBED_DIM)


# speedup vs baseline: 4.6459x; 4.6459x over previous
import functools

import jax
import jax.numpy as jnp
from jax import lax
from jax.experimental import pallas as pl
from jax.experimental.pallas import tpu as pltpu
from jax.experimental.pallas import tpu_sc as plsc

NUM_TOKEN = 100000
EMBED_DIM = 64
GRID_SIZE = 200
B, N = 4096, 200
R = B * N

NW = 32
ROWS_PER_W = R // NW
CHUNK = 512
NG = CHUNK // 128
NCHUNK = ROWS_PER_W // CHUNK
LANES = 16


@functools.partial(
    pl.kernel,
    out_type=jax.ShapeDtypeStruct((R, EMBED_DIM), jnp.float32),
    mesh=plsc.VectorSubcoreMesh(core_axis_name="c", subcore_axis_name="s"),
    compiler_params=pltpu.CompilerParams(use_tc_tiling_on_sc=False),
    scratch_types=[
        pltpu.VMEM((CHUNK,), jnp.int32),
        pltpu.VMEM((NG, 128), jnp.int32),
        pltpu.VMEM((NG, 128), jnp.int32),
        pltpu.VMEM((CHUNK, EMBED_DIM), jnp.float32),
        pltpu.VMEM((CHUNK, EMBED_DIM), jnp.float32),
        pltpu.SemaphoreType.DMA,
    ],
)
def _embed_kernel(x_hbm, tok_hbm, pos_hbm, out_hbm,
                  x_v, tidx_v, pidx_v, trows_v, prows_v, sem):
    wid = lax.axis_index("s") * 2 + lax.axis_index("c")
    base = wid * ROWS_PER_W

    def chunk_body(ci, carry):
        row0 = base + ci * CHUNK
        pltpu.sync_copy(x_hbm.at[pl.ds(row0, CHUNK)], x_v)
        for g in range(CHUNK // LANES):
            a, off = g // 8, (g % 8) * LANES
            xv = x_v[pl.ds(g * LANES, LANES)]
            rowv = row0 + g * LANES + lax.iota(jnp.int32, LANES)
            nv = lax.rem(rowv, N)
            pad = xv == 0
            tidx_v[a, pl.ds(off, LANES)] = jnp.where(
                pad, nv + (NUM_TOKEN - GRID_SIZE), xv)
            pidx_v[a, pl.ds(off, LANES)] = jnp.where(pad, GRID_SIZE, nv)
        copies = []
        for a in range(NG):
            copies.append(pltpu.async_copy(
                tok_hbm.at[tidx_v.at[a]],
                trows_v.at[pl.ds(a * 128, 128)], sem))
            copies.append(pltpu.async_copy(
                pos_hbm.at[pidx_v.at[a]],
                prows_v.at[pl.ds(a * 128, 128)], sem))
        for cp in copies:
            cp.wait()

        def add_body(j, c2):
            for cc in range(EMBED_DIM // LANES):
                sl = pl.ds(cc * LANES, LANES)
                trows_v[j, sl] = trows_v[j, sl] + prows_v[j, sl]
            return c2

        lax.fori_loop(0, CHUNK, add_body, 0)
        pltpu.sync_copy(trows_v, out_hbm.at[pl.ds(row0, CHUNK)])
        return carry

    lax.fori_loop(0, NCHUNK, chunk_body, 0)


def kernel(x, tok_embed, pos_embed):
    out = _embed_kernel(x.reshape(R), tok_embed, pos_embed)
    return out.reshape(B, N, EMBED_DIM)
